# baseline (speedup 1.0000x reference)
import jax
import jax.numpy as jnp
from jax.experimental import pallas as pl
from jax.experimental.pallas import tpu as pltpu

DIM = 2048
ND = 8
RK = 64
NTOK = 16384
BLK = 1024
NBLK = NTOK // BLK
NDRK = ND * RK


def _proj_block_kernel(ids_ref, xlo_ref, xhi_ref, vall_ref, uallt_ref, s_ref,
                       o_ref):
    xlo = xlo_ref[...]
    xhi = xhi_ref[...]
    ids = ids_ref[0, 0, :]
    half = DIM // 2
    z = jnp.dot(xlo, vall_ref[:half, :], preferred_element_type=jnp.float32)
    z = z + jnp.dot(xhi, vall_ref[half:, :], preferred_element_type=jnp.float32)
    col_dom = jax.lax.broadcasted_iota(jnp.int32, (BLK, NDRK), 1) // RK
    keep = (ids[:, None] == col_dom).astype(jnp.float32)
    z = z * keep * s_ref[...]
    o_ref[:, :half] = xlo + jnp.dot(z, uallt_ref[:, :half],
                                    preferred_element_type=jnp.float32)
    o_ref[:, half:] = xhi + jnp.dot(z, uallt_ref[:, half:],
                                    preferred_element_type=jnp.float32)


def _reg_kernel(ids_ref, u_ref, v_ref, s_ref, reg_ref):
    ids = ids_ref[...].astype(jnp.int32)
    row_i = jax.lax.broadcasted_iota(jnp.int32, (RK, RK), 0)
    col_i = jax.lax.broadcasted_iota(jnp.int32, (RK, RK), 1)
    eye = (row_i == col_i).astype(jnp.float32)
    total = jnp.zeros((), dtype=jnp.float32)
    for d in range(ND):
        cnt = jnp.sum((ids == d).astype(jnp.float32))
        any_d = (cnt > 0).astype(jnp.float32)
        ud = u_ref[d]
        vd = v_ref[d]
        gu = jax.lax.dot_general(ud, ud, (((0,), (0,)), ((), ())),
                                 preferred_element_type=jnp.float32)
        gv = jax.lax.dot_general(vd, vd, (((0,), (0,)), ((), ())),
                                 preferred_element_type=jnp.float32)
        reg_d = jnp.mean((gu - eye) ** 2) + jnp.mean((gv - eye) ** 2)
        reg_d = reg_d + 0.1 * jnp.mean(jnp.abs(s_ref[d]))
        total = total + any_d * reg_d
    reg_ref[...] = jnp.full((8, 128), total / ND, dtype=jnp.float32)


def kernel(feats, domain_ids, U, V, s):
    ids3d = domain_ids.reshape(NBLK, 1, BLK)
    v_all = V.transpose(1, 0, 2).reshape(DIM, NDRK)
    u_all_t = U.transpose(0, 2, 1).reshape(NDRK, DIM)
    s_flat = s.reshape(1, NDRK)

    out = pl.pallas_call(
        _proj_block_kernel,
        grid=(NBLK,),
        in_specs=[
            pl.BlockSpec((1, 1, BLK), lambda i: (i, 0, 0)),
            pl.BlockSpec((BLK, DIM // 2), lambda i: (i, 0)),
            pl.BlockSpec((BLK, DIM // 2), lambda i: (i, 1)),
            pl.BlockSpec((DIM, NDRK), lambda i: (0, 0)),
            pl.BlockSpec((NDRK, DIM), lambda i: (0, 0)),
            pl.BlockSpec((1, NDRK), lambda i: (0, 0)),
        ],
        out_specs=pl.BlockSpec((BLK, DIM), lambda i: (i, 0)),
        out_shape=jax.ShapeDtypeStruct((NTOK, DIM), jnp.float32),
    )(ids3d, feats, feats, v_all, u_all_t, s_flat)

    reg = pl.pallas_call(
        _reg_kernel,
        in_specs=[
            pl.BlockSpec((128, 128), lambda: (0, 0)),
            pl.BlockSpec((ND, DIM, RK), lambda: (0, 0, 0)),
            pl.BlockSpec((ND, DIM, RK), lambda: (0, 0, 0)),
            pl.BlockSpec((ND, RK), lambda: (0, 0)),
        ],
        out_specs=pl.BlockSpec((8, 128), lambda: (0, 0)),
        out_shape=jax.ShapeDtypeStruct((8, 128), jnp.float32),
    )(domain_ids.reshape(128, 128), U, V, s)

    return out, reg[0, 0].reshape(1)


# speedup vs baseline: 1.0117x; 1.0117x over previous
import jax
import jax.numpy as jnp
from jax.experimental import pallas as pl
from jax.experimental.pallas import tpu as pltpu

DIM = 2048
ND = 8
RK = 64
NTOK = 16384
BLK = 1024
NBLK = NTOK // BLK
NDRK = ND * RK


def _proj_block_kernel(ids_ref, x_ref, u_ref, v_ref, s_ref, o_ref,
                       vall_s, uallt_s):
    @pl.when(pl.program_id(0) == 0)
    def _build_weights():
        for d in range(ND):
            vall_s[:, d * RK:(d + 1) * RK] = v_ref[d]
            uallt_s[d * RK:(d + 1) * RK, :] = jnp.swapaxes(u_ref[d], 0, 1)

    x = x_ref[...]
    ids = ids_ref[0, 0, :]
    z = jnp.dot(x, vall_s[...], preferred_element_type=jnp.float32)
    col_dom = jax.lax.broadcasted_iota(jnp.int32, (BLK, NDRK), 1) // RK
    keep = (ids[:, None] == col_dom).astype(jnp.float32)
    z = z * keep * s_ref[...]
    proj = jnp.dot(z, uallt_s[...], preferred_element_type=jnp.float32)
    o_ref[...] = x + proj


def _reg_kernel(ids_ref, u_ref, v_ref, s_ref, reg_ref):
    ids = ids_ref[...].astype(jnp.int32)
    row_i = jax.lax.broadcasted_iota(jnp.int32, (RK, RK), 0)
    col_i = jax.lax.broadcasted_iota(jnp.int32, (RK, RK), 1)
    eye = (row_i == col_i).astype(jnp.float32)
    total = jnp.zeros((), dtype=jnp.float32)
    for d in range(ND):
        cnt = jnp.sum((ids == d).astype(jnp.float32))
        any_d = (cnt > 0).astype(jnp.float32)
        ud = u_ref[d]
        vd = v_ref[d]
        gu = jax.lax.dot_general(ud, ud, (((0,), (0,)), ((), ())),
                                 preferred_element_type=jnp.float32)
        gv = jax.lax.dot_general(vd, vd, (((0,), (0,)), ((), ())),
                                 preferred_element_type=jnp.float32)
        reg_d = jnp.mean((gu - eye) ** 2) + jnp.mean((gv - eye) ** 2)
        reg_d = reg_d + 0.1 * jnp.mean(jnp.abs(s_ref[d]))
        total = total + any_d * reg_d
    reg_ref[...] = jnp.full((8, 128), total / ND, dtype=jnp.float32)


def kernel(feats, domain_ids, U, V, s):
    ids3d = domain_ids.reshape(NBLK, 1, BLK)
    s_flat = s.reshape(1, NDRK)

    out = pl.pallas_call(
        _proj_block_kernel,
        grid=(NBLK,),
        in_specs=[
            pl.BlockSpec((1, 1, BLK), lambda i: (i, 0, 0)),
            pl.BlockSpec((BLK, DIM), lambda i: (i, 0)),
            pl.BlockSpec((ND, DIM, RK), lambda i: (0, 0, 0)),
            pl.BlockSpec((ND, DIM, RK), lambda i: (0, 0, 0)),
            pl.BlockSpec((1, NDRK), lambda i: (0, 0)),
        ],
        out_specs=pl.BlockSpec((BLK, DIM), lambda i: (i, 0)),
        out_shape=jax.ShapeDtypeStruct((NTOK, DIM), jnp.float32),
        scratch_shapes=[
            pltpu.VMEM((DIM, NDRK), jnp.float32),
            pltpu.VMEM((NDRK, DIM), jnp.float32),
        ],
    )(ids3d, feats, U, V, s_flat)

    reg = pl.pallas_call(
        _reg_kernel,
        in_specs=[
            pl.BlockSpec((128, 128), lambda: (0, 0)),
            pl.BlockSpec((ND, DIM, RK), lambda: (0, 0, 0)),
            pl.BlockSpec((ND, DIM, RK), lambda: (0, 0, 0)),
            pl.BlockSpec((ND, RK), lambda: (0, 0)),
        ],
        out_specs=pl.BlockSpec((8, 128), lambda: (0, 0)),
        out_shape=jax.ShapeDtypeStruct((8, 128), jnp.float32),
    )(domain_ids.reshape(128, 128), U, V, s)

    return out, reg[0, 0].reshape(1)


# speedup vs baseline: 1.0624x; 1.0501x over previous
import jax
import jax.numpy as jnp
from jax.experimental import pallas as pl
from jax.experimental.pallas import tpu as pltpu

DIM = 2048
ND = 8
RK = 64
NTOK = 16384
BLK = 1024
NBLK = NTOK // BLK
NDRK = ND * RK


def _proj_kernel(ids_ref, x_ref, u_ref, v_ref, s_ref, o_ref, reg_ref,
                 vall_s, uallt_s, cnt_s):
    pid = pl.program_id(0)

    @pl.when(pid == 0)
    def _init():
        for d in range(ND):
            vall_s[:, d * RK:(d + 1) * RK] = v_ref[d]
            uallt_s[d * RK:(d + 1) * RK, :] = jnp.swapaxes(u_ref[d], 0, 1)
        cnt_s[...] = jnp.zeros((1, NDRK), dtype=jnp.float32)

    x = x_ref[...]
    ids = ids_ref[0, 0, :]
    z = jnp.dot(x, vall_s[...], preferred_element_type=jnp.float32)
    col_dom = jax.lax.broadcasted_iota(jnp.int32, (BLK, NDRK), 1) // RK
    keep = (ids[:, None] == col_dom).astype(jnp.float32)
    cnt_s[...] = cnt_s[...] + jnp.sum(keep, axis=0, keepdims=True)
    z = z * keep * s_ref[...]
    proj = jnp.dot(z, uallt_s[...], preferred_element_type=jnp.float32)
    o_ref[...] = x + proj

    @pl.when(pid == NBLK - 1)
    def _reg():
        row_i = jax.lax.broadcasted_iota(jnp.int32, (RK, RK), 0)
        col_i = jax.lax.broadcasted_iota(jnp.int32, (RK, RK), 1)
        eye = (row_i == col_i).astype(jnp.float32)
        total = jnp.zeros((), dtype=jnp.float32)
        for d in range(ND):
            any_d = (cnt_s[0, d * RK] > 0).astype(jnp.float32)
            ud = u_ref[d]
            vd = v_ref[d]
            gu = jax.lax.dot_general(ud, ud, (((0,), (0,)), ((), ())),
                                     preferred_element_type=jnp.float32)
            gv = jax.lax.dot_general(vd, vd, (((0,), (0,)), ((), ())),
                                     preferred_element_type=jnp.float32)
            reg_d = jnp.mean((gu - eye) ** 2) + jnp.mean((gv - eye) ** 2)
            reg_d = reg_d + 0.1 * jnp.mean(jnp.abs(s_ref[0, d * RK:(d + 1) * RK]))
            total = total + any_d * reg_d
        reg_ref[...] = jnp.full((8, 128), total / ND, dtype=jnp.float32)


def kernel(feats, domain_ids, U, V, s):
    ids3d = domain_ids.reshape(NBLK, 1, BLK)
    s_flat = s.reshape(1, NDRK)

    out, reg = pl.pallas_call(
        _proj_kernel,
        grid=(NBLK,),
        in_specs=[
            pl.BlockSpec((1, 1, BLK), lambda i: (i, 0, 0)),
            pl.BlockSpec((BLK, DIM), lambda i: (i, 0)),
            pl.BlockSpec((ND, DIM, RK), lambda i: (0, 0, 0)),
            pl.BlockSpec((ND, DIM, RK), lambda i: (0, 0, 0)),
            pl.BlockSpec((1, NDRK), lambda i: (0, 0)),
        ],
        out_specs=[
            pl.BlockSpec((BLK, DIM), lambda i: (i, 0)),
            pl.BlockSpec((8, 128), lambda i: (0, 0)),
        ],
        out_shape=[
            jax.ShapeDtypeStruct((NTOK, DIM), jnp.float32),
            jax.ShapeDtypeStruct((8, 128), jnp.float32),
        ],
        scratch_shapes=[
            pltpu.VMEM((DIM, NDRK), jnp.float32),
            pltpu.VMEM((NDRK, DIM), jnp.float32),
            pltpu.VMEM((1, NDRK), jnp.float32),
        ],
        compiler_params=pltpu.CompilerParams(
            vmem_limit_bytes=63 * 1024 * 1024),
    )(ids3d, feats, U, V, s_flat)

    return out, reg[0, 0].reshape(1)


# speedup vs baseline: 1.0754x; 1.0123x over previous
import jax
import jax.numpy as jnp
from jax.experimental import pallas as pl
from jax.experimental.pallas import tpu as pltpu

DIM = 2048
ND = 8
RK = 64
NTOK = 16384
BLK = 1024
NBLK = NTOK // BLK
NDRK = ND * RK


def _proj_kernel(ids_ref, x_ref, u_ref, v_ref, s_ref, o_ref, reg_ref,
                 vall_s, uallt_s, cnt_s):
    pid = pl.program_id(0)

    @pl.when(pid == 0)
    def _init():
        for d in range(ND):
            vall_s[:, d * RK:(d + 1) * RK] = v_ref[d]
            sd_col = jnp.swapaxes(s_ref[d:d + 1, :], 0, 1)
            uallt_s[d * RK:(d + 1) * RK, :] = (
                jnp.swapaxes(u_ref[d], 0, 1) * sd_col)
        cnt_s[...] = jnp.zeros((1, NDRK), dtype=jnp.float32)

    x = x_ref[...]
    ids = ids_ref[0, 0, :]
    z = jnp.dot(x, vall_s[...], preferred_element_type=jnp.float32)
    col_dom = jax.lax.broadcasted_iota(jnp.int32, (BLK, NDRK), 1) // RK
    keep = (ids[:, None] == col_dom).astype(jnp.float32)
    cnt_s[...] = cnt_s[...] + jnp.sum(keep, axis=0, keepdims=True)
    proj = jnp.dot(z * keep, uallt_s[...], preferred_element_type=jnp.float32)
    o_ref[...] = x + proj

    @pl.when(pid == NBLK - 1)
    def _reg():
        row_i = jax.lax.broadcasted_iota(jnp.int32, (RK, RK), 0)
        col_i = jax.lax.broadcasted_iota(jnp.int32, (RK, RK), 1)
        eye = (row_i == col_i).astype(jnp.float32)
        total = jnp.zeros((), dtype=jnp.float32)
        for d in range(ND):
            any_d = (cnt_s[0, d * RK] > 0).astype(jnp.float32)
            ud = u_ref[d]
            vd = v_ref[d]
            gu = jax.lax.dot_general(ud, ud, (((0,), (0,)), ((), ())),
                                     preferred_element_type=jnp.float32)
            gv = jax.lax.dot_general(vd, vd, (((0,), (0,)), ((), ())),
                                     preferred_element_type=jnp.float32)
            reg_d = jnp.mean((gu - eye) ** 2) + jnp.mean((gv - eye) ** 2)
            reg_d = reg_d + 0.1 * jnp.mean(jnp.abs(s_ref[d:d + 1, :]))
            total = total + any_d * reg_d
        reg_ref[...] = jnp.full((8, 128), total / ND, dtype=jnp.float32)


def kernel(feats, domain_ids, U, V, s):
    ids3d = domain_ids.reshape(NBLK, 1, BLK)

    out, reg = pl.pallas_call(
        _proj_kernel,
        grid=(NBLK,),
        in_specs=[
            pl.BlockSpec((1, 1, BLK), lambda i: (i, 0, 0)),
            pl.BlockSpec((BLK, DIM), lambda i: (i, 0)),
            pl.BlockSpec((ND, DIM, RK), lambda i: (0, 0, 0)),
            pl.BlockSpec((ND, DIM, RK), lambda i: (0, 0, 0)),
            pl.BlockSpec((ND, RK), lambda i: (0, 0)),
        ],
        out_specs=[
            pl.BlockSpec((BLK, DIM), lambda i: (i, 0)),
            pl.BlockSpec((8, 128), lambda i: (0, 0)),
        ],
        out_shape=[
            jax.ShapeDtypeStruct((NTOK, DIM), jnp.float32),
            jax.ShapeDtypeStruct((8, 128), jnp.float32),
        ],
        scratch_shapes=[
            pltpu.VMEM((DIM, NDRK), jnp.float32),
            pltpu.VMEM((NDRK, DIM), jnp.float32),
            pltpu.VMEM((1, NDRK), jnp.float32),
        ],
        compiler_params=pltpu.CompilerParams(
            vmem_limit_bytes=67043328),
    )(ids3d, feats, U, V, s)

    return out, reg[0, 0].reshape(1)


# speedup vs baseline: 1.0856x; 1.0095x over previous
import jax
import jax.numpy as jnp
from jax.experimental import pallas as pl
from jax.experimental.pallas import tpu as pltpu

DIM = 2048
ND = 8
RK = 64
NTOK = 16384
BLK = 1024
NBLK = NTOK // BLK
NDRK = ND * RK


def _proj_kernel(ids_ref, x_ref, u_ref, v_ref, s_ref, o_ref, reg_ref,
                 vall_s, uallt_s, cnt_s):
    pid = pl.program_id(0)

    @pl.when(pid == 0)
    def _init():
        for d in range(ND):
            vall_s[:, d * RK:(d + 1) * RK] = v_ref[d]
            sd_col = jnp.swapaxes(s_ref[d:d + 1, :], 0, 1)
            uallt_s[d * RK:(d + 1) * RK, :] = (
                jnp.swapaxes(u_ref[d], 0, 1) * sd_col)
        cnt_s[...] = jnp.zeros((1, NDRK), dtype=jnp.float32)

    x = x_ref[...]
    ids = ids_ref[...]
    z = jnp.dot(x, vall_s[...], preferred_element_type=jnp.float32)
    col_dom = jax.lax.broadcasted_iota(jnp.int32, (BLK, NDRK), 1) // RK
    keep = (ids[:, None] == col_dom).astype(jnp.float32)
    cnt_s[...] = cnt_s[...] + jnp.sum(keep, axis=0, keepdims=True)
    proj = jnp.dot(z * keep, uallt_s[...], preferred_element_type=jnp.float32)
    o_ref[...] = x + proj

    @pl.when(pid == NBLK - 1)
    def _reg():
        row_i = jax.lax.broadcasted_iota(jnp.int32, (RK, RK), 0)
        col_i = jax.lax.broadcasted_iota(jnp.int32, (RK, RK), 1)
        eye = (row_i == col_i).astype(jnp.float32)
        total = jnp.zeros((), dtype=jnp.float32)
        for d in range(ND):
            any_d = (cnt_s[0, d * RK] > 0).astype(jnp.float32)
            ud = u_ref[d]
            vd = v_ref[d]
            gu = jax.lax.dot_general(ud, ud, (((0,), (0,)), ((), ())),
                                     preferred_element_type=jnp.float32)
            gv = jax.lax.dot_general(vd, vd, (((0,), (0,)), ((), ())),
                                     preferred_element_type=jnp.float32)
            reg_d = jnp.mean((gu - eye) ** 2) + jnp.mean((gv - eye) ** 2)
            reg_d = reg_d + 0.1 * jnp.mean(jnp.abs(s_ref[d:d + 1, :]))
            total = total + any_d * reg_d
        reg_ref[...] = jnp.full((1,), total / ND, dtype=jnp.float32)


def kernel(feats, domain_ids, U, V, s):
    out, reg = pl.pallas_call(
        _proj_kernel,
        grid=(NBLK,),
        in_specs=[
            pl.BlockSpec((BLK,), lambda i: (i,)),
            pl.BlockSpec((BLK, DIM), lambda i: (i, 0)),
            pl.BlockSpec((ND, DIM, RK), lambda i: (0, 0, 0)),
            pl.BlockSpec((ND, DIM, RK), lambda i: (0, 0, 0)),
            pl.BlockSpec((ND, RK), lambda i: (0, 0)),
        ],
        out_specs=[
            pl.BlockSpec((BLK, DIM), lambda i: (i, 0)),
            pl.BlockSpec((1,), lambda i: (0,)),
        ],
        out_shape=[
            jax.ShapeDtypeStruct((NTOK, DIM), jnp.float32),
            jax.ShapeDtypeStruct((1,), jnp.float32),
        ],
        scratch_shapes=[
            pltpu.VMEM((DIM, NDRK), jnp.float32),
            pltpu.VMEM((NDRK, DIM), jnp.float32),
            pltpu.VMEM((1, NDRK), jnp.float32),
        ],
        compiler_params=pltpu.CompilerParams(
            vmem_limit_bytes=67043328),
    )(domain_ids, feats, U, V, s)

    return out, reg
